# baseline (speedup 1.0000x reference)
import functools

import jax
import jax.numpy as jnp
from jax import lax
from jax.experimental import pallas as pl
from jax.experimental.pallas import tpu as pltpu
from jax.experimental.pallas import tpu_sc as plsc

N = 10000
E = 320000
NC = 2
NS = 16

NPAD = 10240
ROWS_PT = NPAD // NS

C = 128
EPT = 20480
K = EPT // C
DEG_G = 10

AGG_G = 4
SUPK = 32
TRASH = NPAD - 1
HALF = NPAD // 2
ACCR = HALF + 128
LTRASH = ACCR - 1
ARPT = ACCR // NS
CRPT = HALF // NS

_MESH2 = plsc.VectorSubcoreMesh(
    core_axis_name="c", subcore_axis_name="s", num_cores=NC, num_subcores=NS)
_MESH1 = plsc.VectorSubcoreMesh(
    core_axis_name="c", subcore_axis_name="s", num_cores=1, num_subcores=NS)


@functools.partial(
    pl.kernel,
    mesh=_MESH2,
    out_type=jax.ShapeDtypeStruct((2, NPAD), jnp.float32),
    scratch_types=[
        pltpu.VMEM((K, C), jnp.int32),
        pltpu.VMEM((C,), jnp.float32),
        pltpu.VMEM((ROWS_PT,), jnp.float32),
        pltpu.VMEM_SHARED((NPAD,), jnp.float32),
        pltpu.SemaphoreType.DMA,
    ],
)
def _norms_kernel(edges, zeros_pad, out, idx_v, ones_v, row_v, deg_sh, sem):
    cid = lax.axis_index("c")
    sid = lax.axis_index("s")
    pltpu.sync_copy(edges.at[cid, sid], idx_v)
    for i in range(C // 16):
        ones_v[pl.ds(i * 16, 16)] = jnp.ones((16,), jnp.float32)
    pltpu.sync_copy(zeros_pad.at[pl.ds(sid * ROWS_PT, ROWS_PT)],
                    deg_sh.at[pl.ds(sid * ROWS_PT, ROWS_PT)])
    plsc.subcore_barrier()

    def outer(g, carry):
        base = g * DEG_G
        descs = [
            pltpu.async_copy(ones_v, deg_sh.at[idx_v.at[base + i]], sem,
                             add=True)
            for i in range(DEG_G)
        ]
        for d in descs:
            d.wait()
        return carry

    lax.fori_loop(0, K // DEG_G, outer, 0)
    plsc.subcore_barrier()

    pltpu.sync_copy(deg_sh.at[pl.ds(sid * ROWS_PT, ROWS_PT)], row_v)

    def nbody(i, carry):
        x = jnp.maximum(row_v[pl.ds(i * 16, 16)], 1.0)
        yi = jnp.int32(0x5F3759DF) - lax.shift_right_arithmetic(
            lax.bitcast_convert_type(x, jnp.int32), 1)
        y = lax.bitcast_convert_type(yi, jnp.float32)
        for _ in range(3):
            y = y * (1.5 - 0.5 * x * y * y)
        row_v[pl.ds(i * 16, 16)] = y
        return carry

    lax.fori_loop(0, ROWS_PT // 16, nbody, 0)
    pltpu.sync_copy(row_v, out.at[cid, pl.ds(sid * ROWS_PT, ROWS_PT)])


PCAP = 10240
PKC = PCAP // C
GSH = 9


@functools.partial(
    pl.kernel,
    mesh=_MESH2,
    out_type=(
        jax.ShapeDtypeStruct((2, NC * NS, PKC, C), jnp.int32),
        jax.ShapeDtypeStruct((2, NC * NS, PKC, C), jnp.int32),
        jax.ShapeDtypeStruct((NC * NS, 8, 128), jnp.int32),
    ),
    compiler_params=pltpu.CompilerParams(needs_layout_passes=False),
)
def _part_kernel(srcp, dstp, osrc, odst, ocnt):
    cid = lax.axis_index("c")
    sid = lax.axis_index("s")
    p = cid * NS + sid

    def scoped(sv, dv, l0s, l0d, l1s, l1d, cv):
        pltpu.sync_copy(srcp.at[p], sv)
        pltpu.sync_copy(dstp.at[p], dv)
        lanes = lax.iota(jnp.int32, 16)

        def row(r, carry):
            off0, off1 = carry
            for k in range(C // 16):
                s16 = sv[r, pl.ds(k * 16, 16)]
                d16 = dv[r, pl.ds(k * 16, 16)]
                m0 = d16 < HALF
                dloc = jnp.where(m0, d16, d16 - HALF)
                csum = plsc.cumsum(m0.astype(jnp.int32))
                tot0 = jnp.max(csum)
                pos0 = off0 + csum - 1
                plsc.store_scatter(
                    l0s, [lax.shift_right_logical(pos0, 7), pos0 & 127],
                    s16, mask=m0)
                plsc.store_scatter(
                    l0d, [lax.shift_right_logical(pos0, 7), pos0 & 127],
                    dloc, mask=m0)
                m1 = jnp.logical_not(m0)
                pos1 = off1 + (lanes + 1 - csum) - 1
                plsc.store_scatter(
                    l1s, [lax.shift_right_logical(pos1, 7), pos1 & 127],
                    s16, mask=m1)
                plsc.store_scatter(
                    l1d, [lax.shift_right_logical(pos1, 7), pos1 & 127],
                    dloc, mask=m1)
                off0 = off0 + tot0
                off1 = off1 + (16 - tot0)
            return off0, off1

        off0, off1 = lax.fori_loop(0, PKC, row, (jnp.int32(0), jnp.int32(0)))

        zero16 = jnp.zeros((16,), jnp.int32)
        for off, ls, ld, tr in (
            (off0, l0s, l0d, HALF + p),
            (off1, l1s, l1d, HALF + 64 + p),
        ):
            cnt_r = lax.shift_left(
                lax.shift_right_logical(off + 511, GSH), GSH)
            trash = jnp.full((16,), tr, jnp.int32)
            for j in range(33):
                pos = off + j * 16 + lanes
                m = pos < cnt_r
                plsc.store_scatter(
                    ls, [lax.shift_right_logical(pos, 7), pos & 127],
                    zero16, mask=m)
                plsc.store_scatter(
                    ld, [lax.shift_right_logical(pos, 7), pos & 127],
                    trash, mask=m)

        ng0 = lax.shift_right_logical(off0 + 511, GSH)
        ng1 = lax.shift_right_logical(off1 + 511, GSH)
        for r in range(8):
            for k in range(8):
                cv[r, pl.ds(k * 16, 16)] = jnp.zeros((16,), jnp.int32)
        cv[0, pl.ds(0, 16)] = jnp.full((16,), ng0, jnp.int32)
        cv[1, pl.ds(0, 16)] = jnp.full((16,), ng1, jnp.int32)
        pltpu.sync_copy(l0s, osrc.at[0, p])
        pltpu.sync_copy(l0d, odst.at[0, p])
        pltpu.sync_copy(l1s, osrc.at[1, p])
        pltpu.sync_copy(l1d, odst.at[1, p])
        pltpu.sync_copy(cv, ocnt.at[p])

    pl.run_scoped(
        scoped,
        pltpu.VMEM((PKC, C), jnp.int32),
        pltpu.VMEM((PKC, C), jnp.int32),
        pltpu.VMEM((PKC, C), jnp.int32),
        pltpu.VMEM((PKC, C), jnp.int32),
        pltpu.VMEM((PKC, C), jnp.int32),
        pltpu.VMEM((PKC, C), jnp.int32),
        pltpu.VMEM((8, 128), jnp.int32),
    )


def _agg_body(hw, osrc, odst, ocnt, zeros, out, agg_sh, gsems, ssems):
    cid = lax.axis_index("c")
    sid = lax.axis_index("s")

    def scoped(cv, src_v, dst_v, *bufs):
        pltpu.sync_copy(zeros.at[pl.ds(sid * ARPT, ARPT)],
                        agg_sh.at[pl.ds(sid * ARPT, ARPT)])
        plsc.subcore_barrier()

        for h in range(2):
            p = 2 * sid + h
            pltpu.sync_copy(ocnt.at[p], cv)
            pltpu.sync_copy(osrc.at[cid, p], src_v)
            pltpu.sync_copy(odst.at[cid, p], dst_v)
            ng = jnp.max(cv[cid, pl.ds(0, 16)])

            def outer(g, c2):
                base = g * AGG_G
                gd = [
                    pltpu.async_copy(hw.at[src_v.at[base + i]], bufs[i],
                                     gsems[i])
                    for i in range(AGG_G)
                ]
                sd = []
                for i in range(AGG_G):
                    gd[i].wait()
                    sd.append(
                        pltpu.async_copy(bufs[i],
                                         agg_sh.at[dst_v.at[base + i]],
                                         ssems[i], add=True))
                for d in sd:
                    d.wait()
                return c2

            lax.fori_loop(0, ng, outer, 0)

        plsc.subcore_barrier()
        pltpu.sync_copy(agg_sh.at[pl.ds(sid * CRPT, CRPT)],
                        out.at[pl.ds(cid * HALF + sid * CRPT, CRPT)])

    pl.run_scoped(
        scoped,
        pltpu.VMEM((8, 128), jnp.int32),
        pltpu.VMEM((PKC, C), jnp.int32),
        pltpu.VMEM((PKC, C), jnp.int32),
        *[pltpu.VMEM((C, 128), jnp.float32) for _ in range(AGG_G)],
    )


_agg128 = functools.partial(
    pl.kernel,
    mesh=_MESH2,
    out_type=jax.ShapeDtypeStruct((NPAD, 128), jnp.float32),
    scratch_types=[
        pltpu.VMEM_SHARED((ACCR, 128), jnp.float32),
        tuple(pltpu.SemaphoreType.DMA for _ in range(AGG_G)),
        tuple(pltpu.SemaphoreType.DMA for _ in range(AGG_G)),
    ],
    compiler_params=pltpu.CompilerParams(needs_layout_passes=False),
)(_agg_body)


_BM = 1000


def _pro0_body(x_ref, n_ref, w_ref, o_ref):
    o_ref[...] = jnp.dot(x_ref[...] * n_ref[...], w_ref[...],
                         preferred_element_type=jnp.float32)


def _pro0(features, out_norm, W1):
    return pl.pallas_call(
        _pro0_body,
        grid=(N // _BM,),
        in_specs=[
            pl.BlockSpec((_BM, 128), lambda i: (i, 0)),
            pl.BlockSpec((_BM, 1), lambda i: (i, 0)),
            pl.BlockSpec((128, 128), lambda i: (0, 0)),
        ],
        out_specs=pl.BlockSpec((_BM, 128), lambda i: (i, 0)),
        out_shape=jax.ShapeDtypeStruct((N, 128), jnp.float32),
    )(features, out_norm, W1)


def _mid_body(p_ref, in_ref, b_ref, on_ref, w_ref, o_ref):
    t = p_ref[...] * in_ref[...] + b_ref[...]
    t = jnp.maximum(t, 0.0)
    o_ref[...] = jnp.dot(t * on_ref[...], w_ref[...],
                         preferred_element_type=jnp.float32)


def _mid(p, in_norm, b, out_norm, W):
    fi, fo = W.shape
    return pl.pallas_call(
        _mid_body,
        grid=(N // _BM,),
        in_specs=[
            pl.BlockSpec((_BM, fi), lambda i: (i, 0)),
            pl.BlockSpec((_BM, 1), lambda i: (i, 0)),
            pl.BlockSpec((1, fi), lambda i: (0, 0)),
            pl.BlockSpec((_BM, 1), lambda i: (i, 0)),
            pl.BlockSpec((fi, fo), lambda i: (0, 0)),
        ],
        out_specs=pl.BlockSpec((_BM, fo), lambda i: (i, 0)),
        out_shape=jax.ShapeDtypeStruct((N, fo), jnp.float32),
    )(p, in_norm, b, out_norm, W)


def _epi_body(p_ref, in_ref, b_ref, o_ref):
    o_ref[...] = p_ref[...][:, :64] * in_ref[...] + b_ref[...]


def _epi(p, in_norm, b):
    return pl.pallas_call(
        _epi_body,
        grid=(N // _BM,),
        in_specs=[
            pl.BlockSpec((_BM, 128), lambda i: (i, 0)),
            pl.BlockSpec((_BM, 1), lambda i: (i, 0)),
            pl.BlockSpec((1, 64), lambda i: (0, 0)),
        ],
        out_specs=pl.BlockSpec((_BM, 64), lambda i: (i, 0)),
        out_shape=jax.ShapeDtypeStruct((N, 64), jnp.float32),
    )(p, in_norm, b)


def kernel(features, edge_index, W1, b1, W2, b2, W3, b3):
    e32 = edge_index.astype(jnp.int32)
    per_tile = e32.reshape(2, NS, E // NS)
    pad_deg = jnp.pad(per_tile, ((0, 0), (0, 0), (0, EPT - E // NS)),
                      constant_values=TRASH)
    edges_deg = pad_deg.reshape(2, NS, K, C)
    per_pt = e32.reshape(2, NC * NS, E // (NC * NS))
    src_p = jnp.pad(per_pt[0], ((0, 0), (0, PCAP - E // (NC * NS))),
                    constant_values=0).reshape(NC * NS, PKC, C)
    padv = jnp.broadcast_to(
        NPAD + 96 + (jnp.arange(NC * NS, dtype=jnp.int32) % 32)[:, None],
        (NC * NS, PCAP - E // (NC * NS)))
    dst_p = jnp.concatenate([per_pt[1], padv], axis=1).reshape(NC * NS,
                                                              PKC, C)
    zeros_pad = jnp.zeros((NPAD,), jnp.float32)
    zeros128 = jnp.zeros((ACCR, 128), jnp.float32)
    W3p = jnp.pad(W3, ((0, 0), (0, 64)))

    norms = _norms_kernel(edges_deg, zeros_pad)
    out_norm = norms[0, :N].reshape(N, 1)
    in_norm = norms[1, :N].reshape(N, 1)

    lsrc, ldst, lcnt = _part_kernel(src_p, dst_p)

    hw1 = _pro0(features, out_norm, W1)
    p1 = _agg128(hw1, lsrc, ldst, lcnt, zeros128)[:N]
    hw2 = _mid(p1, in_norm, b1.reshape(1, 128), out_norm, W2)
    p2 = _agg128(hw2, lsrc, ldst, lcnt, zeros128)[:N]
    hw3 = _mid(p2, in_norm, b2.reshape(1, 128), out_norm, W3p)
    p3 = _agg128(hw3, lsrc, ldst, lcnt, zeros128)[:N]
    return _epi(p3, in_norm, b3.reshape(1, 64))


# speedup vs baseline: 1.7533x; 1.7533x over previous
import functools

import jax
import jax.numpy as jnp
from jax import lax
from jax.experimental import pallas as pl
from jax.experimental.pallas import tpu as pltpu
from jax.experimental.pallas import tpu_sc as plsc

N = 10000
E = 320000
NC = 2
NS = 16

NPAD = 10240
ROWS_PT = NPAD // NS

C = 128
EPT = 20480
K = EPT // C
DEG_G = 10

AGG_G = 2
SUPK = 32
TRASH = NPAD - 1

_MESH2 = plsc.VectorSubcoreMesh(
    core_axis_name="c", subcore_axis_name="s", num_cores=NC, num_subcores=NS)
_MESH1 = plsc.VectorSubcoreMesh(
    core_axis_name="c", subcore_axis_name="s", num_cores=1, num_subcores=NS)


@functools.partial(
    pl.kernel,
    mesh=_MESH2,
    out_type=jax.ShapeDtypeStruct((2, NPAD), jnp.float32),
    scratch_types=[
        pltpu.VMEM((K, C), jnp.int32),
        pltpu.VMEM((C,), jnp.float32),
        pltpu.VMEM((ROWS_PT,), jnp.float32),
        pltpu.VMEM_SHARED((NPAD,), jnp.float32),
        pltpu.SemaphoreType.DMA,
    ],
)
def _norms_kernel(edges, zeros_pad, out, idx_v, ones_v, row_v, deg_sh, sem):
    cid = lax.axis_index("c")
    sid = lax.axis_index("s")
    pltpu.sync_copy(edges.at[cid, sid], idx_v)
    for i in range(C // 16):
        ones_v[pl.ds(i * 16, 16)] = jnp.ones((16,), jnp.float32)
    pltpu.sync_copy(zeros_pad.at[pl.ds(sid * ROWS_PT, ROWS_PT)],
                    deg_sh.at[pl.ds(sid * ROWS_PT, ROWS_PT)])
    plsc.subcore_barrier()

    def outer(g, carry):
        base = g * DEG_G
        descs = [
            pltpu.async_copy(ones_v, deg_sh.at[idx_v.at[base + i]], sem,
                             add=True)
            for i in range(DEG_G)
        ]
        for d in descs:
            d.wait()
        return carry

    lax.fori_loop(0, K // DEG_G, outer, 0)
    plsc.subcore_barrier()

    pltpu.sync_copy(deg_sh.at[pl.ds(sid * ROWS_PT, ROWS_PT)], row_v)

    def nbody(i, carry):
        x = jnp.maximum(row_v[pl.ds(i * 16, 16)], 1.0)
        yi = jnp.int32(0x5F3759DF) - lax.shift_right_arithmetic(
            lax.bitcast_convert_type(x, jnp.int32), 1)
        y = lax.bitcast_convert_type(yi, jnp.float32)
        for _ in range(3):
            y = y * (1.5 - 0.5 * x * y * y)
        row_v[pl.ds(i * 16, 16)] = y
        return carry

    lax.fori_loop(0, ROWS_PT // 16, nbody, 0)
    pltpu.sync_copy(row_v, out.at[cid, pl.ds(sid * ROWS_PT, ROWS_PT)])


def _agg_body(hw, srcr, dstr, zeros, out, src_v, dst_v, bufs, agg_sh,
              gsems, ssems):
    sid = lax.axis_index("s")
    pltpu.sync_copy(zeros.at[pl.ds(sid * ROWS_PT, ROWS_PT)],
                    agg_sh.at[pl.ds(sid * ROWS_PT, ROWS_PT)])
    plsc.subcore_barrier()

    def sup(sp, carry):
        pltpu.sync_copy(srcr.at[sid, pl.ds(sp * SUPK, SUPK)], src_v)
        pltpu.sync_copy(dstr.at[sid, pl.ds(sp * SUPK, SUPK)], dst_v)

        def outer(g, c2):
            base = g * AGG_G
            gd = []
            for i in range(AGG_G):
                @pl.when(g > 0)
                def _wait_prev(i=i):
                    pltpu.make_async_copy(hw.at[pl.ds(0, C)], bufs[i],
                                          ssems[i]).wait()
                gd.append(
                    pltpu.async_copy(hw.at[src_v.at[base + i]], bufs[i],
                                     gsems[i]))
            for i in range(AGG_G):
                gd[i].wait()
                pltpu.async_copy(bufs[i], agg_sh.at[dst_v.at[base + i]],
                                 ssems[i], add=True)
            return c2

        lax.fori_loop(0, SUPK // AGG_G, outer, 0)
        for i in range(AGG_G):
            pltpu.make_async_copy(hw.at[pl.ds(0, C)], bufs[i],
                                  ssems[i]).wait()
        return carry

    lax.fori_loop(0, K // SUPK, sup, 0)
    plsc.subcore_barrier()
    pltpu.sync_copy(agg_sh.at[pl.ds(sid * ROWS_PT, ROWS_PT)],
                    out.at[pl.ds(sid * ROWS_PT, ROWS_PT)])


_agg128 = functools.partial(
    pl.kernel,
    mesh=_MESH1,
    out_type=jax.ShapeDtypeStruct((NPAD, 128), jnp.float32),
    scratch_types=[
        pltpu.VMEM((SUPK, C), jnp.int32),
        pltpu.VMEM((SUPK, C), jnp.int32),
        tuple(pltpu.VMEM((C, 128), jnp.float32) for _ in range(AGG_G)),
        pltpu.VMEM_SHARED((NPAD, 128), jnp.float32),
        tuple(pltpu.SemaphoreType.DMA for _ in range(AGG_G)),
        tuple(pltpu.SemaphoreType.DMA for _ in range(AGG_G)),
    ],
)(_agg_body)


_BM = 1000


def _pro0_body(x_ref, n_ref, w_ref, o_ref):
    o_ref[...] = jnp.dot(x_ref[...] * n_ref[...], w_ref[...],
                         preferred_element_type=jnp.float32)


def _pro0(features, out_norm, W1):
    return pl.pallas_call(
        _pro0_body,
        grid=(N // _BM,),
        in_specs=[
            pl.BlockSpec((_BM, 128), lambda i: (i, 0)),
            pl.BlockSpec((_BM, 1), lambda i: (i, 0)),
            pl.BlockSpec((128, 128), lambda i: (0, 0)),
        ],
        out_specs=pl.BlockSpec((_BM, 128), lambda i: (i, 0)),
        out_shape=jax.ShapeDtypeStruct((N, 128), jnp.float32),
    )(features, out_norm, W1)


def _mid_body(p_ref, in_ref, b_ref, on_ref, w_ref, o_ref):
    t = p_ref[...] * in_ref[...] + b_ref[...]
    t = jnp.maximum(t, 0.0)
    o_ref[...] = jnp.dot(t * on_ref[...], w_ref[...],
                         preferred_element_type=jnp.float32)


def _mid(p, in_norm, b, out_norm, W):
    fi, fo = W.shape
    return pl.pallas_call(
        _mid_body,
        grid=(N // _BM,),
        in_specs=[
            pl.BlockSpec((_BM, fi), lambda i: (i, 0)),
            pl.BlockSpec((_BM, 1), lambda i: (i, 0)),
            pl.BlockSpec((1, fi), lambda i: (0, 0)),
            pl.BlockSpec((_BM, 1), lambda i: (i, 0)),
            pl.BlockSpec((fi, fo), lambda i: (0, 0)),
        ],
        out_specs=pl.BlockSpec((_BM, fo), lambda i: (i, 0)),
        out_shape=jax.ShapeDtypeStruct((N, fo), jnp.float32),
    )(p, in_norm, b, out_norm, W)


def _epi_body(p_ref, in_ref, b_ref, o_ref):
    o_ref[...] = p_ref[...][:, :64] * in_ref[...] + b_ref[...]


def _epi(p, in_norm, b):
    return pl.pallas_call(
        _epi_body,
        grid=(N // _BM,),
        in_specs=[
            pl.BlockSpec((_BM, 128), lambda i: (i, 0)),
            pl.BlockSpec((_BM, 1), lambda i: (i, 0)),
            pl.BlockSpec((1, 64), lambda i: (0, 0)),
        ],
        out_specs=pl.BlockSpec((_BM, 64), lambda i: (i, 0)),
        out_shape=jax.ShapeDtypeStruct((N, 64), jnp.float32),
    )(p, in_norm, b)


def kernel(features, edge_index, W1, b1, W2, b2, W3, b3):
    e32 = edge_index.astype(jnp.int32)
    per_tile = e32.reshape(2, NS, E // NS)
    pad_deg = jnp.pad(per_tile, ((0, 0), (0, 0), (0, EPT - E // NS)),
                      constant_values=TRASH)
    edges_deg = pad_deg.reshape(2, NS, K, C)
    src_agg = jnp.pad(per_tile[0], ((0, 0), (0, EPT - E // NS)),
                      constant_values=0).reshape(NS, K, C)
    trash_rows = jnp.broadcast_to(
        (N + 8 * jnp.arange(NS, dtype=jnp.int32))[:, None],
        (NS, EPT - E // NS))
    dst_agg = jnp.concatenate([per_tile[1], trash_rows],
                              axis=1).reshape(NS, K, C)
    zeros_pad = jnp.zeros((NPAD,), jnp.float32)
    zeros128 = jnp.zeros((NPAD, 128), jnp.float32)
    W3p = jnp.pad(W3, ((0, 0), (0, 64)))

    norms = _norms_kernel(edges_deg, zeros_pad)
    out_norm = norms[0, :N].reshape(N, 1)
    in_norm = norms[1, :N].reshape(N, 1)

    hw1 = _pro0(features, out_norm, W1)
    p1 = _agg128(hw1, src_agg, dst_agg, zeros128)[:N]
    hw2 = _mid(p1, in_norm, b1.reshape(1, 128), out_norm, W2)
    p2 = _agg128(hw2, src_agg, dst_agg, zeros128)[:N]
    hw3 = _mid(p2, in_norm, b2.reshape(1, 128), out_norm, W3p)
    p3 = _agg128(hw3, src_agg, dst_agg, zeros128)[:N]
    return _epi(p3, in_norm, b3.reshape(1, 64))
